# baseline (speedup 1.0000x reference)
import jax
import jax.numpy as jnp
from jax import lax
from jax.experimental import pallas as pl
from jax.experimental.pallas import tpu as pltpu

_X_CHUNK = 1000
_BM_AGG = 200
_ADJ_BUFFERS = 4


def _layer_body(x_hbm, w_v, b_v, al_v, adj_hbm, o_hbm, y_scr, x_bufs, sem):
    n = y_scr.shape[0]
    nf = n // _X_CHUNK

    def _start(k):
        pltpu.make_async_copy(
            x_hbm.at[pl.ds(k * _X_CHUNK, _X_CHUNK)],
            x_bufs.at[k % 2], sem.at[k % 2]).start()

    _start(0)
    for k in range(nf):
        if k + 1 < nf:
            _start(k + 1)
        pltpu.make_async_copy(
            x_hbm.at[pl.ds(k * _X_CHUNK, _X_CHUNK)],
            x_bufs.at[k % 2], sem.at[k % 2]).wait()
        y_scr[pl.ds(k * _X_CHUNK, _X_CHUNK), :] = lax.dot_general(
            x_bufs[k % 2], w_v[...], (((1,), (1,)), ((), ())),
            preferred_element_type=jnp.float32)

    def _agg_body(a_ref, o_ref):
        acc = lax.dot_general(
            a_ref[...], y_scr[...], (((1,), (0,)), ((), ())),
            preferred_element_type=jnp.float32)
        h = acc + b_v[...]
        alpha = al_v[0, 0]
        o_ref[...] = jnp.where(h >= 0.0, h, alpha * h).astype(o_ref.dtype)

    pltpu.emit_pipeline(
        _agg_body,
        grid=(n // _BM_AGG,),
        in_specs=[pl.BlockSpec(
            (_BM_AGG, n), lambda i: (i, 0),
            pipeline_mode=pl.Buffered(
                buffer_count=_ADJ_BUFFERS, use_lookahead=True))],
        out_specs=[pl.BlockSpec((_BM_AGG, o_hbm.shape[1]), lambda i: (i, 0))],
    )(adj_hbm, o_hbm)


def _gcn_layer(x, adj2d, w, b, alpha, out_dtype=jnp.float32):
    n, d_in = x.shape
    d_out = w.shape[0]
    return pl.pallas_call(
        _layer_body,
        in_specs=[
            pl.BlockSpec(memory_space=pltpu.HBM),
            pl.BlockSpec(memory_space=pltpu.VMEM),
            pl.BlockSpec(memory_space=pltpu.VMEM),
            pl.BlockSpec(memory_space=pltpu.VMEM),
            pl.BlockSpec(memory_space=pltpu.HBM),
        ],
        out_specs=pl.BlockSpec(memory_space=pltpu.HBM),
        out_shape=jax.ShapeDtypeStruct((n, d_out), out_dtype),
        scratch_shapes=[
            pltpu.VMEM((n, d_out), jnp.float32),
            pltpu.VMEM((2, _X_CHUNK, d_in), x.dtype),
            pltpu.SemaphoreType.DMA((2,)),
        ],
    )(x, w, b.reshape(1, -1), alpha.reshape(1, 1), adj2d)


def kernel(features, seq1, adj, b1, W1, a1, b2, W2, a2, sparse):
    del seq1, sparse
    x = features[0]
    adj2d = adj[0]
    h1 = _gcn_layer(x, adj2d, W1, b1, a1, out_dtype=jnp.bfloat16)
    h2 = _gcn_layer(h1, adj2d, W2.astype(jnp.bfloat16), b2, a2)
    return h2[None]
